# baseline (speedup 1.0000x reference)
import functools

import jax
import jax.numpy as jnp
from jax import lax
from jax.experimental import pallas as pl
from jax.experimental.pallas import tpu as pltpu
from jax.experimental.pallas import tpu_sc as plsc

NPTS = 8192
LANES = 16
NC = 2
NS = 16
NW = NC * NS
CHUNK = NPTS // NW
QV = CHUNK // LANES
SB = 2
QVSB = QV // SB


def _halfnorm_pass(x_ref, y_ref, z_ref, th_ref):
    def body(v, carry):
        sl = pl.ds(v * LANES, LANES)
        x = x_ref[sl]
        y = y_ref[sl]
        z = z_ref[sl]
        th_ref[sl] = 0.5 * (x * x + y * y + z * z)
        return carry

    lax.fori_loop(0, NPTS // LANES, body, 0, unroll=4)


def _nn_pass(qx_ref, qy_ref, qz_ref, dbx_ref, dby_ref, dbz_ref, th_ref,
             out_ref, qbase):
    for sb in range(SB):
        qxs = []
        qys = []
        qzs = []
        for j in range(QVSB):
            sl = pl.ds(qbase + (sb * QVSB + j) * LANES, LANES)
            qxs.append(qx_ref[sl])
            qys.append(qy_ref[sl])
            qzs.append(qz_ref[sl])

        neg_inf = jnp.full((LANES,), -jnp.inf, dtype=jnp.float32)
        init = tuple(neg_inf for _ in range(QVSB))

        def body(v, accs):
            sl = pl.ds(v * LANES, LANES)
            dx16 = dbx_ref[sl]
            dy16 = dby_ref[sl]
            dz16 = dbz_ref[sl]
            th16 = th_ref[sl]
            accs = list(accs)
            for k in range(LANES):
                dxs = jnp.full((LANES,), dx16[k], dtype=jnp.float32)
                dys = jnp.full((LANES,), dy16[k], dtype=jnp.float32)
                dzs = jnp.full((LANES,), dz16[k], dtype=jnp.float32)
                ths = jnp.full((LANES,), th16[k], dtype=jnp.float32)
                for j in range(QVSB):
                    s = qxs[j] * dxs + qys[j] * dys + qzs[j] * dzs - ths
                    accs[j] = jnp.maximum(accs[j], s)
            return tuple(accs)

        accs = lax.fori_loop(0, NPTS // LANES, body, init)

        for j in range(QVSB):
            qn = qxs[j] * qxs[j] + qys[j] * qys[j] + qzs[j] * qzs[j]
            d2 = qn - (accs[j] + accs[j])
            d2 = jnp.maximum(d2, 0.0)
            out_ref[pl.ds((sb * QVSB + j) * LANES, LANES)] = d2


def _sc_body(px_hbm, py_hbm, pz_hbm, tx_hbm, ty_hbm, tz_hbm,
             p2t_hbm, t2p_hbm,
             px_v, py_v, pz_v, tx_v, ty_v, tz_v, thp_v, tht_v,
             outa_v, outb_v):
    wid = lax.axis_index("s") * NC + lax.axis_index("c")
    qbase = wid * CHUNK

    pltpu.sync_copy(px_hbm, px_v)
    pltpu.sync_copy(py_hbm, py_v)
    pltpu.sync_copy(pz_hbm, pz_v)
    pltpu.sync_copy(tx_hbm, tx_v)
    pltpu.sync_copy(ty_hbm, ty_v)
    pltpu.sync_copy(tz_hbm, tz_v)

    _halfnorm_pass(tx_v, ty_v, tz_v, tht_v)
    _halfnorm_pass(px_v, py_v, pz_v, thp_v)

    _nn_pass(px_v, py_v, pz_v, tx_v, ty_v, tz_v, tht_v, outa_v, qbase)
    _nn_pass(tx_v, ty_v, tz_v, px_v, py_v, pz_v, thp_v, outb_v, qbase)

    pltpu.sync_copy(outa_v, p2t_hbm.at[pl.ds(qbase, CHUNK)])
    pltpu.sync_copy(outb_v, t2p_hbm.at[pl.ds(qbase, CHUNK)])


_sc_chamfer = functools.partial(
    pl.kernel,
    out_type=(
        jax.ShapeDtypeStruct((NPTS,), jnp.float32),
        jax.ShapeDtypeStruct((NPTS,), jnp.float32),
    ),
    mesh=plsc.VectorSubcoreMesh(core_axis_name="c", subcore_axis_name="s"),
    scratch_types=[pltpu.VMEM((NPTS,), jnp.float32)] * 8
    + [pltpu.VMEM((CHUNK,), jnp.float32)] * 2,
)(_sc_body)


def _combine_body(a_ref, b_ref, out_ref):
    suma = jnp.sum(jnp.sqrt(jnp.maximum(a_ref[:, :], 0.0)))
    sumb = jnp.sum(jnp.sqrt(jnp.maximum(b_ref[:, :], 0.0)))
    out_ref[0, 0] = suma * (1.0 / NPTS) + sumb * (1.0 / NPTS)


def _combine(a, b):
    return pl.pallas_call(
        _combine_body,
        out_specs=pl.BlockSpec(memory_space=pltpu.SMEM),
        out_shape=jax.ShapeDtypeStruct((1, 1), jnp.float32),
    )(a.reshape(8, NPTS // 8), b.reshape(8, NPTS // 8))


@jax.jit
def kernel(pred, target):
    px, py, pz = pred[:, 0], pred[:, 1], pred[:, 2]
    tx, ty, tz = target[:, 0], target[:, 1], target[:, 2]
    p2t_d2, t2p_d2 = _sc_chamfer(px, py, pz, tx, ty, tz)
    return _combine(p2t_d2, t2p_d2)[0, 0]


# speedup vs baseline: 1.2894x; 1.2894x over previous
import functools

import jax
import jax.numpy as jnp
from jax import lax
from jax.experimental import pallas as pl
from jax.experimental.pallas import tpu as pltpu
from jax.experimental.pallas import tpu_sc as plsc

NPTS = 8192
LANES = 16
NC = 2
NS = 16
NW = NC * NS
CHUNK = NPTS // NW
QV = CHUNK // LANES
SB = 2
QVSB = QV // SB


_GDN = lax.GatherDimensionNumbers(
    offset_dims=(), collapsed_slice_dims=(0,), start_index_map=(0,)
)


def _lane_bcast(v, k):
    idx = jnp.full((LANES, 1), k, dtype=jnp.int32)
    return lax.gather(
        v, idx, _GDN, slice_sizes=(1,),
        mode=lax.GatherScatterMode.PROMISE_IN_BOUNDS,
    )


def _halfnorm_pass(x_ref, y_ref, z_ref, th_ref):
    def body(v, carry):
        sl = pl.ds(v * LANES, LANES)
        x = x_ref[sl]
        y = y_ref[sl]
        z = z_ref[sl]
        th_ref[sl] = 0.5 * (x * x + y * y + z * z)
        return carry

    lax.fori_loop(0, NPTS // LANES, body, 0, unroll=4)


def _nn_pass(qx_ref, qy_ref, qz_ref, dbx_ref, dby_ref, dbz_ref, th_ref,
             out_ref, qbase):
    def sub_block(sb, carry):
        qxs = []
        qys = []
        qzs = []
        for j in range(QVSB):
            sl = pl.ds(qbase + sb * (QVSB * LANES) + j * LANES, LANES)
            qxs.append(qx_ref[sl])
            qys.append(qy_ref[sl])
            qzs.append(qz_ref[sl])

        neg_inf = jnp.full((LANES,), -jnp.inf, dtype=jnp.float32)
        init = tuple(neg_inf for _ in range(QVSB))

        def body(v, accs):
            sl = pl.ds(v * LANES, LANES)
            dx16 = dbx_ref[sl]
            dy16 = dby_ref[sl]
            dz16 = dbz_ref[sl]
            th16 = th_ref[sl]
            accs = list(accs)
            for k in range(LANES):
                dxs = _lane_bcast(dx16, k)
                dys = _lane_bcast(dy16, k)
                dzs = _lane_bcast(dz16, k)
                ths = _lane_bcast(th16, k)
                for j in range(QVSB):
                    s = qxs[j] * dxs + qys[j] * dys + qzs[j] * dzs - ths
                    accs[j] = jnp.maximum(accs[j], s)
            return tuple(accs)

        accs = lax.fori_loop(0, NPTS // LANES, body, init)

        for j in range(QVSB):
            qn = qxs[j] * qxs[j] + qys[j] * qys[j] + qzs[j] * qzs[j]
            d2 = qn - (accs[j] + accs[j])
            d2 = jnp.maximum(d2, 0.0)
            out_ref[pl.ds(sb * (QVSB * LANES) + j * LANES, LANES)] = d2
        return carry

    lax.fori_loop(0, SB, sub_block, 0)


def _sc_body(px_hbm, py_hbm, pz_hbm, tx_hbm, ty_hbm, tz_hbm,
             p2t_hbm, t2p_hbm,
             px_v, py_v, pz_v, tx_v, ty_v, tz_v, thp_v, tht_v,
             outa_v, outb_v):
    wid = lax.axis_index("s") * NC + lax.axis_index("c")
    qbase = wid * CHUNK

    pltpu.sync_copy(px_hbm, px_v)
    pltpu.sync_copy(py_hbm, py_v)
    pltpu.sync_copy(pz_hbm, pz_v)
    pltpu.sync_copy(tx_hbm, tx_v)
    pltpu.sync_copy(ty_hbm, ty_v)
    pltpu.sync_copy(tz_hbm, tz_v)

    _halfnorm_pass(tx_v, ty_v, tz_v, tht_v)
    _halfnorm_pass(px_v, py_v, pz_v, thp_v)

    _nn_pass(px_v, py_v, pz_v, tx_v, ty_v, tz_v, tht_v, outa_v, qbase)
    _nn_pass(tx_v, ty_v, tz_v, px_v, py_v, pz_v, thp_v, outb_v, qbase)

    pltpu.sync_copy(outa_v, p2t_hbm.at[pl.ds(qbase, CHUNK)])
    pltpu.sync_copy(outb_v, t2p_hbm.at[pl.ds(qbase, CHUNK)])


_sc_chamfer = functools.partial(
    pl.kernel,
    out_type=(
        jax.ShapeDtypeStruct((NPTS,), jnp.float32),
        jax.ShapeDtypeStruct((NPTS,), jnp.float32),
    ),
    mesh=plsc.VectorSubcoreMesh(core_axis_name="c", subcore_axis_name="s"),
    scratch_types=[pltpu.VMEM((NPTS,), jnp.float32)] * 8
    + [pltpu.VMEM((CHUNK,), jnp.float32)] * 2,
)(_sc_body)


def _combine_body(a_ref, b_ref, out_ref):
    suma = jnp.sum(jnp.sqrt(jnp.maximum(a_ref[:, :], 0.0)))
    sumb = jnp.sum(jnp.sqrt(jnp.maximum(b_ref[:, :], 0.0)))
    out_ref[0, 0] = suma * (1.0 / NPTS) + sumb * (1.0 / NPTS)


def _combine(a, b):
    return pl.pallas_call(
        _combine_body,
        out_specs=pl.BlockSpec(memory_space=pltpu.SMEM),
        out_shape=jax.ShapeDtypeStruct((1, 1), jnp.float32),
    )(a.reshape(8, NPTS // 8), b.reshape(8, NPTS // 8))


@jax.jit
def kernel(pred, target):
    px, py, pz = pred[:, 0], pred[:, 1], pred[:, 2]
    tx, ty, tz = target[:, 0], target[:, 1], target[:, 2]
    p2t_d2, t2p_d2 = _sc_chamfer(px, py, pz, tx, ty, tz)
    return _combine(p2t_d2, t2p_d2)[0, 0]


# speedup vs baseline: 1.2901x; 1.0006x over previous
import functools

import jax
import jax.numpy as jnp
from jax import lax
from jax.experimental import pallas as pl
from jax.experimental.pallas import tpu as pltpu
from jax.experimental.pallas import tpu_sc as plsc

NPTS = 8192
LANES = 16
NC = 2
NS = 16
NW = NC * NS
CHUNK = NPTS // NW
QV = CHUNK // LANES
SB = 2
QVSB = QV // SB


_GDN = lax.GatherDimensionNumbers(
    offset_dims=(), collapsed_slice_dims=(0,), start_index_map=(0,)
)


def _lane_bcast(v, k):
    idx = jnp.full((LANES, 1), k, dtype=jnp.int32)
    return lax.gather(
        v, idx, _GDN, slice_sizes=(1,),
        mode=lax.GatherScatterMode.PROMISE_IN_BOUNDS,
    )


def _halfnorm_pass(x_ref, y_ref, z_ref, th_ref):
    def body(v, carry):
        sl = pl.ds(v * LANES, LANES)
        x = x_ref[sl]
        y = y_ref[sl]
        z = z_ref[sl]
        th_ref[sl] = 0.5 * (x * x + y * y + z * z)
        return carry

    lax.fori_loop(0, NPTS // LANES, body, 0, unroll=4)


def _nn_pass(qx_ref, qy_ref, qz_ref, dbx_ref, dby_ref, dbz_ref, th_ref,
             out_ref, qbase):
    def sub_block(sb, carry):
        qxs = []
        qys = []
        qzs = []
        for j in range(QVSB):
            sl = pl.ds(qbase + sb * (QVSB * LANES) + j * LANES, LANES)
            qxs.append(qx_ref[sl])
            qys.append(qy_ref[sl])
            qzs.append(qz_ref[sl])

        neg_inf = jnp.full((LANES,), -jnp.inf, dtype=jnp.float32)
        init = tuple(neg_inf for _ in range(QVSB))

        @plsc.parallel_loop(0, NPTS // LANES, carry=init, unroll=2)
        def accs(v, accs):
            sl = pl.ds(v * LANES, LANES)
            dx16 = dbx_ref[sl]
            dy16 = dby_ref[sl]
            dz16 = dbz_ref[sl]
            th16 = th_ref[sl]
            accs = list(accs)
            for k in range(LANES):
                dxs = _lane_bcast(dx16, k)
                dys = _lane_bcast(dy16, k)
                dzs = _lane_bcast(dz16, k)
                ths = _lane_bcast(th16, k)
                for j in range(QVSB):
                    s = qxs[j] * dxs + qys[j] * dys + qzs[j] * dzs - ths
                    accs[j] = jnp.maximum(accs[j], s)
            return tuple(accs)

        for j in range(QVSB):
            qn = qxs[j] * qxs[j] + qys[j] * qys[j] + qzs[j] * qzs[j]
            d2 = qn - (accs[j] + accs[j])
            d2 = jnp.maximum(d2, 0.0)
            out_ref[pl.ds(sb * (QVSB * LANES) + j * LANES, LANES)] = d2
        return carry

    lax.fori_loop(0, SB, sub_block, 0)


def _sc_body(px_hbm, py_hbm, pz_hbm, tx_hbm, ty_hbm, tz_hbm,
             p2t_hbm, t2p_hbm,
             px_v, py_v, pz_v, tx_v, ty_v, tz_v, thp_v, tht_v,
             outa_v, outb_v):
    wid = lax.axis_index("s") * NC + lax.axis_index("c")
    qbase = wid * CHUNK

    pltpu.sync_copy(px_hbm, px_v)
    pltpu.sync_copy(py_hbm, py_v)
    pltpu.sync_copy(pz_hbm, pz_v)
    pltpu.sync_copy(tx_hbm, tx_v)
    pltpu.sync_copy(ty_hbm, ty_v)
    pltpu.sync_copy(tz_hbm, tz_v)

    _halfnorm_pass(tx_v, ty_v, tz_v, tht_v)
    _halfnorm_pass(px_v, py_v, pz_v, thp_v)

    _nn_pass(px_v, py_v, pz_v, tx_v, ty_v, tz_v, tht_v, outa_v, qbase)
    _nn_pass(tx_v, ty_v, tz_v, px_v, py_v, pz_v, thp_v, outb_v, qbase)

    pltpu.sync_copy(outa_v, p2t_hbm.at[pl.ds(qbase, CHUNK)])
    pltpu.sync_copy(outb_v, t2p_hbm.at[pl.ds(qbase, CHUNK)])


_sc_chamfer = functools.partial(
    pl.kernel,
    out_type=(
        jax.ShapeDtypeStruct((NPTS,), jnp.float32),
        jax.ShapeDtypeStruct((NPTS,), jnp.float32),
    ),
    mesh=plsc.VectorSubcoreMesh(core_axis_name="c", subcore_axis_name="s"),
    scratch_types=[pltpu.VMEM((NPTS,), jnp.float32)] * 8
    + [pltpu.VMEM((CHUNK,), jnp.float32)] * 2,
)(_sc_body)


def _combine_body(a_ref, b_ref, out_ref):
    suma = jnp.sum(jnp.sqrt(jnp.maximum(a_ref[:, :], 0.0)))
    sumb = jnp.sum(jnp.sqrt(jnp.maximum(b_ref[:, :], 0.0)))
    out_ref[0, 0] = suma * (1.0 / NPTS) + sumb * (1.0 / NPTS)


def _combine(a, b):
    return pl.pallas_call(
        _combine_body,
        out_specs=pl.BlockSpec(memory_space=pltpu.SMEM),
        out_shape=jax.ShapeDtypeStruct((1, 1), jnp.float32),
    )(a.reshape(8, NPTS // 8), b.reshape(8, NPTS // 8))


@jax.jit
def kernel(pred, target):
    px, py, pz = pred[:, 0], pred[:, 1], pred[:, 2]
    tx, ty, tz = target[:, 0], target[:, 1], target[:, 2]
    p2t_d2, t2p_d2 = _sc_chamfer(px, py, pz, tx, ty, tz)
    return _combine(p2t_d2, t2p_d2)[0, 0]


# speedup vs baseline: 5.4158x; 4.1979x over previous
import functools

import jax
import jax.numpy as jnp
from jax import lax
from jax.experimental import pallas as pl
from jax.experimental.pallas import tpu as pltpu
from jax.experimental.pallas import tpu_sc as plsc

NPTS = 8192
LANES = 16
NC = 2
NS = 16
NW = NC * NS
CHUNK = NPTS // NW
QV = CHUNK // LANES
SB = 2
QVSB = QV // SB


_GDN = lax.GatherDimensionNumbers(
    offset_dims=(), collapsed_slice_dims=(0,), start_index_map=(0,)
)


def _lane_bcast(v, k):
    idx = jnp.full((LANES,), k, dtype=jnp.int32).reshape(LANES, 1)
    return lax.gather(
        v, idx, _GDN, slice_sizes=(1,),
        mode=lax.GatherScatterMode.PROMISE_IN_BOUNDS,
    )


def _halfnorm_pass(x_ref, y_ref, z_ref, th_ref):
    def body(v, carry):
        sl = pl.ds(v * LANES, LANES)
        x = x_ref[sl]
        y = y_ref[sl]
        z = z_ref[sl]
        th_ref[sl] = 0.5 * (x * x + y * y + z * z)
        return carry

    lax.fori_loop(0, NPTS // LANES, body, 0, unroll=4)


def _nn_pass(qx_ref, qy_ref, qz_ref, dbx_ref, dby_ref, dbz_ref, th_ref,
             out_ref, qbase):
    def sub_block(sb, carry):
        qxs = []
        qys = []
        qzs = []
        for j in range(QVSB):
            sl = pl.ds(qbase + sb * (QVSB * LANES) + j * LANES, LANES)
            qxs.append(qx_ref[sl])
            qys.append(qy_ref[sl])
            qzs.append(qz_ref[sl])

        neg_inf = jnp.full((LANES,), -jnp.inf, dtype=jnp.float32)
        init = tuple(neg_inf for _ in range(QVSB))

        @plsc.parallel_loop(0, NPTS // LANES, carry=init, unroll=1)
        def accs(v, accs):
            sl = pl.ds(v * LANES, LANES)
            dx16 = dbx_ref[sl]
            dy16 = dby_ref[sl]
            dz16 = dbz_ref[sl]
            th16 = th_ref[sl]

            def kbody(k, accs):
                accs = list(accs)
                for kk in range(2):
                    lane = k * 2 + kk
                    dxs = _lane_bcast(dx16, lane)
                    dys = _lane_bcast(dy16, lane)
                    dzs = _lane_bcast(dz16, lane)
                    ths = _lane_bcast(th16, lane)
                    for j in range(QVSB):
                        s = qxs[j] * dxs + qys[j] * dys + qzs[j] * dzs - ths
                        accs[j] = jnp.maximum(accs[j], s)
                return tuple(accs)

            return lax.fori_loop(0, LANES // 2, kbody, tuple(accs))

        for j in range(QVSB):
            qn = qxs[j] * qxs[j] + qys[j] * qys[j] + qzs[j] * qzs[j]
            d2 = qn - (accs[j] + accs[j])
            d2 = jnp.maximum(d2, 0.0)
            out_ref[pl.ds(sb * (QVSB * LANES) + j * LANES, LANES)] = d2
        return carry

    lax.fori_loop(0, SB, sub_block, 0)


def _sc_body(px_hbm, py_hbm, pz_hbm, tx_hbm, ty_hbm, tz_hbm,
             p2t_hbm, t2p_hbm,
             px_v, py_v, pz_v, tx_v, ty_v, tz_v, thp_v, tht_v,
             outa_v, outb_v):
    wid = lax.axis_index("s") * NC + lax.axis_index("c")
    qbase = wid * CHUNK

    pltpu.sync_copy(px_hbm, px_v)
    pltpu.sync_copy(py_hbm, py_v)
    pltpu.sync_copy(pz_hbm, pz_v)
    pltpu.sync_copy(tx_hbm, tx_v)
    pltpu.sync_copy(ty_hbm, ty_v)
    pltpu.sync_copy(tz_hbm, tz_v)

    _halfnorm_pass(tx_v, ty_v, tz_v, tht_v)
    _halfnorm_pass(px_v, py_v, pz_v, thp_v)

    _nn_pass(px_v, py_v, pz_v, tx_v, ty_v, tz_v, tht_v, outa_v, qbase)
    _nn_pass(tx_v, ty_v, tz_v, px_v, py_v, pz_v, thp_v, outb_v, qbase)

    pltpu.sync_copy(outa_v, p2t_hbm.at[pl.ds(qbase, CHUNK)])
    pltpu.sync_copy(outb_v, t2p_hbm.at[pl.ds(qbase, CHUNK)])


_sc_chamfer = functools.partial(
    pl.kernel,
    out_type=(
        jax.ShapeDtypeStruct((NPTS,), jnp.float32),
        jax.ShapeDtypeStruct((NPTS,), jnp.float32),
    ),
    mesh=plsc.VectorSubcoreMesh(core_axis_name="c", subcore_axis_name="s"),
    scratch_types=[pltpu.VMEM((NPTS,), jnp.float32)] * 8
    + [pltpu.VMEM((CHUNK,), jnp.float32)] * 2,
)(_sc_body)


def _combine_body(a_ref, b_ref, out_ref):
    suma = jnp.sum(jnp.sqrt(jnp.maximum(a_ref[:, :], 0.0)))
    sumb = jnp.sum(jnp.sqrt(jnp.maximum(b_ref[:, :], 0.0)))
    out_ref[0, 0] = suma * (1.0 / NPTS) + sumb * (1.0 / NPTS)


def _combine(a, b):
    return pl.pallas_call(
        _combine_body,
        out_specs=pl.BlockSpec(memory_space=pltpu.SMEM),
        out_shape=jax.ShapeDtypeStruct((1, 1), jnp.float32),
    )(a.reshape(8, NPTS // 8), b.reshape(8, NPTS // 8))


@jax.jit
def kernel(pred, target):
    px, py, pz = pred[:, 0], pred[:, 1], pred[:, 2]
    tx, ty, tz = target[:, 0], target[:, 1], target[:, 2]
    p2t_d2, t2p_d2 = _sc_chamfer(px, py, pz, tx, ty, tz)
    return _combine(p2t_d2, t2p_d2)[0, 0]


# speedup vs baseline: 23.8866x; 4.4105x over previous
import functools

import jax
import jax.numpy as jnp
from jax import lax
from jax.experimental import pallas as pl
from jax.experimental.pallas import tpu as pltpu
from jax.experimental.pallas import tpu_sc as plsc

N = 8192
M = 8192
CSPLIT = 1024
BJ = 512
NJ = (M - CSPLIT) // BJ

LANES = 16
NC = 2
NS = 16
NW = NC * NS
CHUNK_A = N // NW
CHUNK_B = CSPLIT // NW

_GDN = lax.GatherDimensionNumbers(
    offset_dims=(), collapsed_slice_dims=(0,), start_index_map=(0,)
)


def _lane_bcast(v, k):
    idx = jnp.full((LANES,), k, dtype=jnp.int32).reshape(LANES, 1)
    return lax.gather(
        v, idx, _GDN, slice_sizes=(1,),
        mode=lax.GatherScatterMode.PROMISE_IN_BOUNDS,
    )


def _halfnorm_pass(x_ref, y_ref, z_ref, th_ref, count):
    def body(v, carry):
        sl = pl.ds(v * LANES, LANES)
        x = x_ref[sl]
        y = y_ref[sl]
        z = z_ref[sl]
        th_ref[sl] = 0.5 * (x * x + y * y + z * z)
        return carry

    lax.fori_loop(0, count // LANES, body, 0, unroll=4)


def _nn_pass(qx_ref, qy_ref, qz_ref, dbx_ref, dby_ref, dbz_ref, th_ref,
             out_ref, qbase, obase, nq, db_n, qvsb):
    nsb = nq // (qvsb * LANES)

    def sub_block(sb, carry):
        qxs = []
        qys = []
        qzs = []
        for j in range(qvsb):
            sl = pl.ds(qbase + sb * (qvsb * LANES) + j * LANES, LANES)
            qxs.append(qx_ref[sl])
            qys.append(qy_ref[sl])
            qzs.append(qz_ref[sl])

        neg_inf = jnp.full((LANES,), -jnp.inf, dtype=jnp.float32)
        init = tuple(neg_inf for _ in range(qvsb))

        @plsc.parallel_loop(0, db_n // LANES, carry=init, unroll=1)
        def accs(v, accs):
            sl = pl.ds(v * LANES, LANES)
            dx16 = dbx_ref[sl]
            dy16 = dby_ref[sl]
            dz16 = dbz_ref[sl]
            th16 = th_ref[sl]

            def kbody(k, accs):
                accs = list(accs)
                for kk in range(2):
                    lane = k * 2 + kk
                    dxs = _lane_bcast(dx16, lane)
                    dys = _lane_bcast(dy16, lane)
                    dzs = _lane_bcast(dz16, lane)
                    ths = _lane_bcast(th16, lane)
                    for j in range(qvsb):
                        s = qxs[j] * dxs + qys[j] * dys + qzs[j] * dzs - ths
                        accs[j] = jnp.maximum(accs[j], s)
                return tuple(accs)

            return lax.fori_loop(0, LANES // 2, kbody, tuple(accs))

        for j in range(qvsb):
            qn = qxs[j] * qxs[j] + qys[j] * qys[j] + qzs[j] * qzs[j]
            d2 = qn - (accs[j] + accs[j])
            d2 = jnp.maximum(d2, 0.0)
            out_ref[pl.ds(obase + sb * (qvsb * LANES) + j * LANES, LANES)] = d2
        return carry

    lax.fori_loop(0, nsb, sub_block, 0)


def _sc_body(px_hbm, py_hbm, pz_hbm, tx_hbm, ty_hbm, tz_hbm,
             prow_hbm, tcol_hbm,
             px_v, py_v, pz_v, tx_v, ty_v, tz_v, thp_v, tht_v,
             outa_v, outb_v):
    wid = lax.axis_index("s") * NC + lax.axis_index("c")

    pltpu.sync_copy(px_hbm, px_v)
    pltpu.sync_copy(py_hbm, py_v)
    pltpu.sync_copy(pz_hbm, pz_v)
    pltpu.sync_copy(tx_hbm, tx_v)
    pltpu.sync_copy(ty_hbm, ty_v)
    pltpu.sync_copy(tz_hbm, tz_v)

    _halfnorm_pass(tx_v, ty_v, tz_v, tht_v, CSPLIT)
    _halfnorm_pass(px_v, py_v, pz_v, thp_v, N)

    _nn_pass(px_v, py_v, pz_v, tx_v, ty_v, tz_v, tht_v, outa_v,
             wid * CHUNK_A, 0, CHUNK_A, CSPLIT, 8)
    _nn_pass(tx_v, ty_v, tz_v, px_v, py_v, pz_v, thp_v, outb_v,
             wid * CHUNK_B, 0, CHUNK_B, N, 2)

    pltpu.sync_copy(outa_v, prow_hbm.at[pl.ds(wid * CHUNK_A, CHUNK_A)])
    pltpu.sync_copy(outb_v, tcol_hbm.at[pl.ds(wid * CHUNK_B, CHUNK_B)])


_sc_chamfer = functools.partial(
    pl.kernel,
    out_type=(
        jax.ShapeDtypeStruct((N,), jnp.float32),
        jax.ShapeDtypeStruct((CSPLIT,), jnp.float32),
    ),
    mesh=plsc.VectorSubcoreMesh(core_axis_name="c", subcore_axis_name="s"),
    scratch_types=[
        pltpu.VMEM((N,), jnp.float32),
        pltpu.VMEM((N,), jnp.float32),
        pltpu.VMEM((N,), jnp.float32),
        pltpu.VMEM((CSPLIT,), jnp.float32),
        pltpu.VMEM((CSPLIT,), jnp.float32),
        pltpu.VMEM((CSPLIT,), jnp.float32),
        pltpu.VMEM((N,), jnp.float32),
        pltpu.VMEM((CSPLIT,), jnp.float32),
        pltpu.VMEM((CHUNK_A,), jnp.float32),
        pltpu.VMEM((CHUNK_B,), jnp.float32),
    ],
)(_sc_body)


def _tc_body(pred_ref, tgt_ref, rowout_ref, colmin_ref, rowmin_ref):
    j = pl.program_id(0)

    px = pred_ref[:, 0:1]
    py = pred_ref[:, 1:2]
    pz = pred_ref[:, 2:3]
    tx = tgt_ref[0:1, :]
    ty = tgt_ref[1:2, :]
    tz = tgt_ref[2:3, :]

    dx = px - tx
    dy = py - ty
    dz = pz - tz
    d2 = dz * dz + (dy * dy + dx * dx)

    folded = jnp.minimum(
        jnp.minimum(d2[:, 0:128], d2[:, 128:256]),
        jnp.minimum(d2[:, 256:384], d2[:, 384:512]),
    )
    colmin_ref[0:1, :] = jnp.min(d2, axis=0, keepdims=True)

    @pl.when(j == 0)
    def _init():
        rowmin_ref[:, :] = folded

    @pl.when(j > 0)
    def _acc():
        rowmin_ref[:, :] = jnp.minimum(rowmin_ref[:, :], folded)

    @pl.when(j == NJ - 1)
    def _final():
        rowout_ref[:, :] = rowmin_ref[:, :]


def _tc_part(pred, tgt_t):
    return pl.pallas_call(
        _tc_body,
        grid=(NJ,),
        in_specs=[
            pl.BlockSpec((N, 3), lambda j: (0, 0)),
            pl.BlockSpec((3, BJ), lambda j: (0, j + CSPLIT // BJ)),
        ],
        out_specs=[
            pl.BlockSpec((N, 128), lambda j: (0, 0)),
            pl.BlockSpec((1, BJ), lambda j: (0, j)),
        ],
        out_shape=[
            jax.ShapeDtypeStruct((N, 128), jnp.float32),
            jax.ShapeDtypeStruct((1, M - CSPLIT), jnp.float32),
        ],
        scratch_shapes=[pltpu.VMEM((N, 128), jnp.float32)],
    )(pred, tgt_t)


def _combine_body(tcrow_ref, scrow_ref, tccol_ref, sccol_ref, out_ref):
    rowmin = jnp.min(tcrow_ref[:, :], axis=1, keepdims=True)
    rowmin = jnp.minimum(rowmin, scrow_ref[:, :])
    rowsum = jnp.sum(jnp.sqrt(jnp.maximum(rowmin, 0.0)))
    colsum = jnp.sum(jnp.sqrt(jnp.maximum(tccol_ref[:, :], 0.0)))
    colsum += jnp.sum(jnp.sqrt(jnp.maximum(sccol_ref[:, :], 0.0)))
    out_ref[0, 0] = rowsum * (1.0 / N) + colsum * (1.0 / M)


def _combine(tcrow, scrow, tccol, sccol):
    return pl.pallas_call(
        _combine_body,
        out_specs=pl.BlockSpec(memory_space=pltpu.SMEM),
        out_shape=jax.ShapeDtypeStruct((1, 1), jnp.float32),
    )(tcrow, scrow.reshape(N, 1), tccol, sccol.reshape(1, CSPLIT))


@jax.jit
def kernel(pred, target):
    px, py, pz = pred[:, 0], pred[:, 1], pred[:, 2]
    tx, ty, tz = target[:, 0], target[:, 1], target[:, 2]
    sc_rowmin, sc_colmin = _sc_chamfer(
        px, py, pz, tx[:CSPLIT], ty[:CSPLIT], tz[:CSPLIT]
    )
    tc_rowmin, tc_colmin = _tc_part(pred, target.T)
    return _combine(tc_rowmin, sc_rowmin, tc_colmin, sc_colmin)[0, 0]
